# baseline (speedup 1.0000x reference)
import functools

import jax
import jax.numpy as jnp
from jax import lax
from jax.experimental import pallas as pl
from jax.experimental.pallas import tpu as pltpu
from jax.experimental.pallas import tpu_sc as plsc

VOCAB = 100000
EMBED = 64
BATCH = 1024
CTX = 50

NC, NS, L = 2, 16, 16
NW = NC * NS
ROWS_PER_W = BATCH // NW
IDX_PER_W = ROWS_PER_W * CTX
GCHUNK = 80
NCHUNK = IDX_PER_W // GCHUNK

VBLK = 2048
NV = (VOCAB + VBLK - 1) // VBLK


def _sc_hidden_body(table_hbm, idx_hbm, hid_hbm, idx_v, rows_v, hid_v, sem):
    wid = lax.axis_index("s") * NC + lax.axis_index("c")
    pltpu.sync_copy(idx_hbm.at[wid], idx_v)
    handles = []
    for c in range(NCHUNK):
        handles.append(
            pltpu.async_copy(
                table_hbm.at[idx_v.at[c]],
                rows_v.at[pl.ds(c * GCHUNK, GCHUNK)],
                sem,
            )
        )
    for h in handles:
        h.wait()

    inv = jnp.float32(1.0 / CTX)

    def row_body(r, carry):
        def j_body(j, accs):
            b = r * CTX + j
            return tuple(
                accs[c] + rows_v[b, pl.ds(c * L, L)] for c in range(EMBED // L)
            )

        zeros = tuple(
            jnp.zeros((L,), jnp.float32) for _ in range(EMBED // L)
        )
        accs = lax.fori_loop(0, CTX, j_body, zeros)
        for c in range(EMBED // L):
            hid_v[r, pl.ds(c * L, L)] = accs[c] * inv
        return carry

    lax.fori_loop(0, ROWS_PER_W, row_body, 0)
    pltpu.sync_copy(hid_v, hid_hbm.at[pl.ds(wid * ROWS_PER_W, ROWS_PER_W)])


@functools.cache
def _sc_hidden():
    return pl.kernel(
        _sc_hidden_body,
        out_type=jax.ShapeDtypeStruct((BATCH, EMBED), jnp.float32),
        mesh=plsc.VectorSubcoreMesh(
            core_axis_name="c",
            subcore_axis_name="s",
            num_cores=NC,
            num_subcores=NS,
        ),
        scratch_types=[
            pltpu.VMEM((NCHUNK, GCHUNK), jnp.int32),
            pltpu.VMEM((IDX_PER_W, EMBED), jnp.float32),
            pltpu.VMEM((ROWS_PER_W, EMBED), jnp.float32),
            pltpu.SemaphoreType.DMA,
        ],
        compiler_params=pltpu.CompilerParams(use_tc_tiling_on_sc=False),
    )


def _logits_tile(hid_ref, w_ref, b_ref):
    h = hid_ref[...].astype(jnp.bfloat16)
    w = w_ref[...].astype(jnp.bfloat16)
    logits = lax.dot_general(
        h, w, (((1,), (1,)), ((), ())), preferred_element_type=jnp.float32
    )
    return logits + b_ref[...]


def _pass_a_body(hid_ref, w_ref, b_ref, m_ref, s_ref):
    vb = pl.program_id(0)

    @pl.when(vb == 0)
    def _():
        m_ref[...] = jnp.full((BATCH, 1), -1e30, jnp.float32)
        s_ref[...] = jnp.zeros((BATCH, 1), jnp.float32)

    logits = _logits_tile(hid_ref, w_ref, b_ref)
    tmax = jnp.max(logits, axis=1, keepdims=True)
    tsum = jnp.sum(jnp.exp(logits - tmax), axis=1, keepdims=True)
    m_old = m_ref[...]
    m_new = jnp.maximum(m_old, tmax)
    s_ref[...] = s_ref[...] * jnp.exp(m_old - m_new) + tsum * jnp.exp(
        tmax - m_new
    )
    m_ref[...] = m_new


def _pass_b_body(hid_ref, w_ref, b_ref, m_ref, s_ref, out_ref):
    logits = _logits_tile(hid_ref, w_ref, b_ref)
    out_ref[...] = logits - (m_ref[...] + jnp.log(s_ref[...]))


def _tc_log_softmax(hidden, out_W, out_b2d, interpret=False):
    hid_spec = pl.BlockSpec((BATCH, EMBED), lambda v: (0, 0))
    w_spec = pl.BlockSpec((VBLK, EMBED), lambda v: (v, 0))
    b_spec = pl.BlockSpec((1, VBLK), lambda v: (0, v))
    ms_spec = pl.BlockSpec((BATCH, 1), lambda v: (0, 0))

    m, s = pl.pallas_call(
        _pass_a_body,
        grid=(NV,),
        in_specs=[hid_spec, w_spec, b_spec],
        out_specs=[ms_spec, ms_spec],
        out_shape=[
            jax.ShapeDtypeStruct((BATCH, 1), jnp.float32),
            jax.ShapeDtypeStruct((BATCH, 1), jnp.float32),
        ],
        compiler_params=pltpu.CompilerParams(
            dimension_semantics=("arbitrary",)
        ),
        interpret=interpret,
    )(hidden, out_W, out_b2d)

    out = pl.pallas_call(
        _pass_b_body,
        grid=(NV,),
        in_specs=[hid_spec, w_spec, b_spec, ms_spec, ms_spec],
        out_specs=pl.BlockSpec((BATCH, VBLK), lambda v: (0, v)),
        out_shape=jax.ShapeDtypeStruct((BATCH, VOCAB), jnp.float32),
        compiler_params=pltpu.CompilerParams(
            dimension_semantics=("arbitrary",)
        ),
        interpret=interpret,
    )(hidden, out_W, out_b2d, m, s)
    return out


MBLK = 2048
MNV = (VOCAB + MBLK - 1) // MBLK
EDGE = VOCAB - (MNV - 1) * MBLK
NBUF = 3


def _mdma(bufs, out_hbm, sems, v_static_size, i, col, size):
    return pltpu.make_async_copy(
        bufs.at[i, :, pl.ds(0, size)],
        out_hbm.at[:, pl.ds(col, size)],
        sems.at[i],
    )


def _pass_b_manual_probe(hid_ref, w_ref, b_ref, out_hbm, bufs, sems):
    v = pl.program_id(0)
    i = lax.rem(v, NBUF)

    @pl.when(v >= NBUF)
    def _():
        _mdma(bufs, out_hbm, sems, None, i, (v - NBUF) * MBLK, MBLK).wait()

    h = hid_ref[...].astype(jnp.bfloat16)
    w = w_ref[...].astype(jnp.bfloat16)
    logits = lax.dot_general(
        h, w, (((1,), (1,)), ((), ())), preferred_element_type=jnp.float32
    )
    bufs[i, ...] = logits + b_ref[0]

    _mdma(bufs, out_hbm, sems, None, i, v * MBLK, MBLK).start()

    @pl.when(v == MNV - 2)
    def _():
        for vv in range(MNV - 1 - NBUF, MNV - 1):
            _mdma(bufs, out_hbm, sems, None, vv % NBUF, vv * MBLK, MBLK).wait()


PROWS = 16
PN = BATCH // PROWS
PBUF = 4


def _rowpanel_probe_body(b_ref, out_hbm, buf0, buf1, buf2, buf3, sems):
    v = pl.program_id(0)
    bufs = (buf0, buf1, buf2, buf3)
    val = jnp.broadcast_to(b_ref[0, 0:1], (PROWS, VOCAB))
    for k in range(PBUF):
        @pl.when(lax.rem(v, PBUF) == k)
        def _(k=k):
            buf = bufs[k]

            @pl.when(v >= PBUF)
            def _():
                pltpu.make_async_copy(
                    buf, out_hbm.at[pl.ds((v - PBUF) * PROWS, PROWS)],
                    sems.at[k],
                ).wait()

            buf[...] = val
            pltpu.async_copy(
                buf, out_hbm.at[pl.ds(v * PROWS, PROWS)], sems.at[k],
                priority=k,
            )

    @pl.when(v == PN - 1)
    def _():
        for vv in range(PN - PBUF, PN):
            pltpu.make_async_copy(
                bufs[vv % PBUF], out_hbm.at[pl.ds(vv * PROWS, PROWS)],
                sems.at[vv % PBUF],
            ).wait()


VPAD = NV * VBLK - VOCAB


def kernel(inputs, emb_table, out_W, out_b):
    idx = inputs.reshape(NW, NCHUNK, GCHUNK)
    hidden = _sc_hidden()(emb_table, idx)
    w_pad = jnp.pad(out_W, ((0, VPAD), (0, 0)))
    b_pad = jnp.pad(out_b, (0, VPAD), constant_values=-1e30)
    return _tc_log_softmax(hidden, w_pad, b_pad.reshape(1, NV * VBLK))


# speedup vs baseline: 1.0323x; 1.0323x over previous
import functools

import jax
import jax.numpy as jnp
from jax import lax
from jax.experimental import pallas as pl
from jax.experimental.pallas import tpu as pltpu
from jax.experimental.pallas import tpu_sc as plsc

VOCAB = 100000
EMBED = 64
BATCH = 1024
CTX = 50

NC, NS, L = 2, 16, 16
NW = NC * NS
ROWS_PER_W = BATCH // NW
IDX_PER_W = ROWS_PER_W * CTX
GCHUNK = 80
NCHUNK = IDX_PER_W // GCHUNK

VBLK = 2048
NV = (VOCAB + VBLK - 1) // VBLK


def _sc_hidden_body(table_hbm, idx_hbm, hid_hbm, idx_v, rows_v, hid_v, sem):
    wid = lax.axis_index("s") * NC + lax.axis_index("c")
    pltpu.sync_copy(idx_hbm.at[wid], idx_v)
    handles = []
    for c in range(NCHUNK):
        handles.append(
            pltpu.async_copy(
                table_hbm.at[idx_v.at[c]],
                rows_v.at[pl.ds(c * GCHUNK, GCHUNK)],
                sem,
            )
        )
    for h in handles:
        h.wait()

    inv = jnp.float32(1.0 / CTX)

    def row_body(r, carry):
        def j_body(j, accs):
            b = r * CTX + j
            return tuple(
                accs[c] + rows_v[b, pl.ds(c * L, L)] for c in range(EMBED // L)
            )

        zeros = tuple(
            jnp.zeros((L,), jnp.float32) for _ in range(EMBED // L)
        )
        accs = lax.fori_loop(0, CTX, j_body, zeros)
        for c in range(EMBED // L):
            hid_v[r, pl.ds(c * L, L)] = accs[c] * inv
        return carry

    lax.fori_loop(0, ROWS_PER_W, row_body, 0)
    pltpu.sync_copy(hid_v, hid_hbm.at[pl.ds(wid * ROWS_PER_W, ROWS_PER_W)])


@functools.cache
def _sc_hidden():
    return pl.kernel(
        _sc_hidden_body,
        out_type=jax.ShapeDtypeStruct((BATCH, EMBED), jnp.float32),
        mesh=plsc.VectorSubcoreMesh(
            core_axis_name="c",
            subcore_axis_name="s",
            num_cores=NC,
            num_subcores=NS,
        ),
        scratch_types=[
            pltpu.VMEM((NCHUNK, GCHUNK), jnp.int32),
            pltpu.VMEM((IDX_PER_W, EMBED), jnp.float32),
            pltpu.VMEM((ROWS_PER_W, EMBED), jnp.float32),
            pltpu.SemaphoreType.DMA,
        ],
        compiler_params=pltpu.CompilerParams(use_tc_tiling_on_sc=False),
    )


def _fused_body(hid_ref, w_ref, b_ref, raw_ref, m_ref, s_ref):
    vb = pl.program_id(0)

    @pl.when(vb == 0)
    def _():
        m_ref[...] = jnp.full((BATCH, 1), -1e30, jnp.float32)
        s_ref[...] = jnp.zeros((BATCH, 1), jnp.float32)

    h = hid_ref[...].astype(jnp.bfloat16)
    w = w_ref[...].astype(jnp.bfloat16)
    logits = lax.dot_general(
        h, w, (((1,), (1,)), ((), ())), preferred_element_type=jnp.float32
    )
    logits = logits + b_ref[...]
    raw_ref[...] = logits.astype(jnp.bfloat16)
    col = lax.broadcasted_iota(jnp.int32, (1, VBLK), 1) + vb * VBLK
    logits = jnp.where(col < VOCAB, logits, -jnp.inf)
    tmax = jnp.max(logits, axis=1, keepdims=True)
    tsum = jnp.sum(jnp.exp(logits - tmax), axis=1, keepdims=True)
    m_old = m_ref[...]
    m_new = jnp.maximum(m_old, tmax)
    s_ref[...] = s_ref[...] * jnp.exp(m_old - m_new) + tsum * jnp.exp(
        tmax - m_new
    )
    m_ref[...] = m_new


def kernel(inputs, emb_table, out_W, out_b):
    idx = inputs.reshape(NW, NCHUNK, GCHUNK)
    hidden = _sc_hidden()(emb_table, idx)

    hid_spec = pl.BlockSpec((BATCH, EMBED), lambda v: (0, 0))
    w_spec = pl.BlockSpec((VBLK, EMBED), lambda v: (v, 0))
    b_spec = pl.BlockSpec((1, VBLK), lambda v: (0, v))
    ms_spec = pl.BlockSpec((BATCH, 1), lambda v: (0, 0))

    raw, m, sum_exp = pl.pallas_call(
        _fused_body,
        grid=(NV,),
        in_specs=[hid_spec, w_spec, b_spec],
        out_specs=[
            pl.BlockSpec((BATCH, VBLK), lambda v: (0, v)),
            ms_spec,
            ms_spec,
        ],
        out_shape=[
            jax.ShapeDtypeStruct((BATCH, VOCAB), jnp.bfloat16),
            jax.ShapeDtypeStruct((BATCH, 1), jnp.float32),
            jax.ShapeDtypeStruct((BATCH, 1), jnp.float32),
        ],
        compiler_params=pltpu.CompilerParams(
            dimension_semantics=("arbitrary",)
        ),
    )(hidden, out_W, out_b.reshape(1, VOCAB))

    lse = m + jnp.log(sum_exp)
    return raw.astype(jnp.float32) - lse
